# baseline (speedup 1.0000x reference)
import functools

import jax
import jax.numpy as jnp
from jax import lax
from jax.experimental import pallas as pl
from jax.experimental.pallas import tpu as pltpu
from jax.experimental.pallas import tpu_sc as plsc

N_NODE = 10000
E = 320000
D = 128

NC = 2
NS = 16
LANES = 16
NPAD = 10240
ROWS_PER_TILE = NPAD // NS
CH = 80
EDGES_PER_TILE = E // (NC * NS)
N_CHUNK = EDGES_PER_TILE // CH
IB = 25
NBLK = N_CHUNK // IB
ZR = 32


def _sc_aggregate(h_user, h_item, src_uci, dst_uci, src_icu, dst_icu):
    mesh = plsc.VectorSubcoreMesh(
        core_axis_name="c", subcore_axis_name="s",
        num_cores=NC, num_subcores=NS)

    out_type = (
        jax.ShapeDtypeStruct((NC, NPAD, D), jnp.float32),
        jax.ShapeDtypeStruct((NC, NPAD, LANES), jnp.float32),
        jax.ShapeDtypeStruct((NC, NPAD, D), jnp.float32),
        jax.ShapeDtypeStruct((NC, NPAD, LANES), jnp.float32),
    )

    @functools.partial(
        pl.kernel, mesh=mesh, out_type=out_type,
        compiler_params=pltpu.CompilerParams(use_tc_tiling_on_sc=False),
        scratch_types=[
            pltpu.VMEM_SHARED((NPAD, D), jnp.float32),
            pltpu.VMEM_SHARED((NPAD, LANES), jnp.float32),
            pltpu.VMEM((IB, CH), jnp.int32),
            pltpu.VMEM((IB, CH), jnp.int32),
            pltpu.VMEM((2, CH, D), jnp.float32),
            pltpu.VMEM((CH, LANES), jnp.float32),
            pltpu.VMEM((ZR, D), jnp.float32),
            pltpu.VMEM((ZR, LANES), jnp.float32),
            pltpu.SemaphoreType.DMA,
            pltpu.SemaphoreType.DMA,
            pltpu.SemaphoreType.DMA,
            pltpu.SemaphoreType.DMA,
            pltpu.SemaphoreType.DMA,
        ],
    )
    def sc_kernel(hu, hi, s_uci, d_uci, s_icu, d_icu,
                  agg_item, cnt_item, agg_user, cnt_user,
                  agg_sh, cnt_sh, sidx_v, didx_v, rows_v,
                  ones_v, zagg_v, zcnt_v, gsemA, gsemB, ssemA, ssemB, isem):
        c = lax.axis_index("c")
        s = lax.axis_index("s")
        row0 = s * ROWS_PER_TILE

        zeros16 = jnp.zeros((LANES,), jnp.float32)
        ones16 = jnp.ones((LANES,), jnp.float32)

        def _fill_rows(ref, nrows, val):
            def row_body(r, _):
                def col_body(cc, __):
                    ref[r, pl.ds(cc * LANES, LANES)] = val
                    return __
                return lax.fori_loop(0, ref.shape[1] // LANES, col_body, _)
            lax.fori_loop(0, nrows, row_body, 0)

        _fill_rows(zagg_v, ZR, zeros16)
        _fill_rows(zcnt_v, ZR, zeros16)
        _fill_rows(ones_v, CH, ones16)

        for h, src_e, dst_e, agg_out, cnt_out in (
                (hu, s_uci, d_uci, agg_item, cnt_item),
                (hi, s_icu, d_icu, agg_user, cnt_user)):
            zcopies = []
            for j in range(ROWS_PER_TILE // ZR):
                zcopies.append(pltpu.async_copy(
                    zagg_v, agg_sh.at[pl.ds(row0 + j * ZR, ZR)], gsemA))
                zcopies.append(pltpu.async_copy(
                    zcnt_v, cnt_sh.at[pl.ds(row0 + j * ZR, ZR)], gsemA))
            for dsc in zcopies:
                dsc.wait()
            plsc.subcore_barrier()

            r0 = c * (E // NC // CH) + s * N_CHUNK

            def rows_scatter(q, i, p):
                return (rows_v.at[p], agg_sh.at[didx_v.at[i]])

            def ones_scatter(q, i):
                return (ones_v, cnt_sh.at[didx_v.at[i]])

            def block_body(blk, carry):
                q = 0
                brow = r0 + blk * IB
                pltpu.sync_copy(src_e.at[pl.ds(brow, IB)], sidx_v)
                pltpu.sync_copy(dst_e.at[pl.ds(brow, IB)], didx_v)

                pltpu.async_copy(h.at[sidx_v.at[0]], rows_v.at[0], gsemA)

                def drain_chunk(i, p, sem):
                    rs, rd = rows_scatter(q, i, p)
                    pltpu.make_async_copy(rs, rd, sem).wait()
                    os_, od = ones_scatter(q, i)
                    pltpu.make_async_copy(os_, od, sem).wait()

                def issue_chunk(i, p, sem):
                    rs, rd = rows_scatter(q, i, p)
                    pltpu.async_copy(rs, rd, sem, add=True)
                    os_, od = ones_scatter(q, i)
                    pltpu.async_copy(os_, od, sem, add=True)

                def pair_body(k, carry2):
                    i0 = 2 * k
                    i1 = 2 * k + 1

                    @pl.when(k >= 1)
                    def _():
                        drain_chunk(i0 - 1, 1, ssemB)

                    pltpu.async_copy(h.at[sidx_v.at[i1]],
                                     rows_v.at[1], gsemB)
                    pltpu.make_async_copy(h.at[sidx_v.at[i0]],
                                          rows_v.at[0], gsemA).wait()
                    issue_chunk(i0, 0, ssemA)
                    pltpu.make_async_copy(h.at[sidx_v.at[i1]],
                                          rows_v.at[1], gsemB).wait()
                    issue_chunk(i1, 1, ssemB)
                    drain_chunk(i0, 0, ssemA)
                    pltpu.async_copy(h.at[sidx_v.at[i0 + 2]],
                                     rows_v.at[0], gsemA)
                    return carry2

                lax.fori_loop(0, IB // 2, pair_body, 0)

                drain_chunk(jnp.int32(IB - 2), 1, ssemB)
                pltpu.make_async_copy(h.at[sidx_v.at[IB - 1]],
                                      rows_v.at[0], gsemA).wait()
                issue_chunk(jnp.int32(IB - 1), 0, ssemA)
                drain_chunk(jnp.int32(IB - 1), 0, ssemA)
                return carry

            lax.fori_loop(0, NBLK, block_body, 0)
            plsc.subcore_barrier()

            pltpu.sync_copy(agg_sh.at[pl.ds(row0, ROWS_PER_TILE)],
                            agg_out.at[c, pl.ds(row0, ROWS_PER_TILE)])
            pltpu.sync_copy(cnt_sh.at[pl.ds(row0, ROWS_PER_TILE)],
                            cnt_out.at[c, pl.ds(row0, ROWS_PER_TILE)])
            plsc.subcore_barrier()

    return sc_kernel(h_user, h_item, src_uci, dst_uci, src_icu, dst_icu)


_BLK = 1000
_GRID = N_NODE // _BLK


def _pre_body(xu_ref, xi_ref, wu_ref, wi_ref, hu_ref, hi_ref):
    dn = (((1,), (1,)), ((), ()))
    hu_ref[...] = lax.dot_general(xu_ref[...], wu_ref[...], dn,
                                  preferred_element_type=jnp.float32)
    hi_ref[...] = lax.dot_general(xi_ref[...], wi_ref[...], dn,
                                  preferred_element_type=jnp.float32)


def _tc_pre(x_user, x_item, w_uci, w_icu):
    blk_x = pl.BlockSpec((_BLK, D), lambda i: (i, 0))
    blk_w = pl.BlockSpec((D, D), lambda i: (0, 0))
    return pl.pallas_call(
        _pre_body,
        grid=(_GRID,),
        in_specs=[blk_x, blk_x, blk_w, blk_w],
        out_specs=[blk_x, blk_x],
        out_shape=[jax.ShapeDtypeStruct((N_NODE, D), jnp.float32)] * 2,
    )(x_user, x_item, w_uci, w_icu)


def _post_body(xu_ref, wu_ref, bsu_ref, bnu_ref, pu_ref, cu_ref,
               xi_ref, wi_ref, bsi_ref, bni_ref, pi_ref, ci_ref,
               ou_ref, oi_ref):
    dn = (((1,), (1,)), ((), ()))
    for x_ref, w_ref, bs_ref, bn_ref, p_ref, c_ref, o_ref in (
            (xu_ref, wu_ref, bsu_ref, bnu_ref, pu_ref, cu_ref, ou_ref),
            (xi_ref, wi_ref, bsi_ref, bni_ref, pi_ref, ci_ref, oi_ref)):
        agg = p_ref[0] + p_ref[1]
        cnt = jnp.maximum(c_ref[0, :, :1] + c_ref[1, :, :1], 1.0)
        self_t = lax.dot_general(x_ref[...], w_ref[...], dn,
                                 preferred_element_type=jnp.float32)
        o_ref[...] = jnp.maximum(
            self_t + bs_ref[...] + agg / cnt + bn_ref[...], 0.0)


def _tc_post(x_user, w_self_user, b_self_user, b_neigh_icu, p_user, c_user,
             x_item, w_self_item, b_self_item, b_neigh_uci, p_item, c_item):
    blk_x = pl.BlockSpec((_BLK, D), lambda i: (i, 0))
    blk_w = pl.BlockSpec((D, D), lambda i: (0, 0))
    blk_b = pl.BlockSpec((1, D), lambda i: (0, 0))
    blk_p = pl.BlockSpec((NC, _BLK, D), lambda i: (0, i, 0))
    blk_c = pl.BlockSpec((NC, _BLK, LANES), lambda i: (0, i, 0))
    return pl.pallas_call(
        _post_body,
        grid=(_GRID,),
        in_specs=[blk_x, blk_w, blk_b, blk_b, blk_p, blk_c,
                  blk_x, blk_w, blk_b, blk_b, blk_p, blk_c],
        out_specs=[blk_x, blk_x],
        out_shape=[jax.ShapeDtypeStruct((N_NODE, D), jnp.float32)] * 2,
    )(x_user, w_self_user, b_self_user, b_neigh_icu, p_user, c_user,
      x_item, w_self_item, b_self_item, b_neigh_uci, p_item, c_item)


def kernel(x_user, x_item, edge_index_user_clicks_item,
           edge_index_item_rev_clicks_user,
           W_self_user, b_self_user, W_self_item, b_self_item,
           W_neigh_uci, b_neigh_uci, W_neigh_icu, b_neigh_icu):
    h_user, h_item = _tc_pre(x_user, x_item, W_neigh_uci, W_neigh_icu)

    agg_item, cnt_item, agg_user, cnt_user = _sc_aggregate(
        h_user, h_item,
        edge_index_user_clicks_item[0].reshape(E // CH, CH),
        edge_index_user_clicks_item[1].reshape(E // CH, CH),
        edge_index_item_rev_clicks_user[0].reshape(E // CH, CH),
        edge_index_item_rev_clicks_user[1].reshape(E // CH, CH))

    out_user, out_item = _tc_post(
        x_user, W_self_user, b_self_user.reshape(1, D), b_neigh_icu.reshape(1, D),
        agg_user, cnt_user,
        x_item, W_self_item, b_self_item.reshape(1, D), b_neigh_uci.reshape(1, D),
        agg_item, cnt_item)
    return out_user, out_item


# speedup vs baseline: 1.2639x; 1.2639x over previous
import functools

import jax
import jax.numpy as jnp
from jax import lax
from jax.experimental import pallas as pl
from jax.experimental.pallas import tpu as pltpu
from jax.experimental.pallas import tpu_sc as plsc

N_NODE = 10000
E = 320000
D = 128

NC = 2
NS = 16
LANES = 16
NPAD = 10240
ROWS_PER_TILE = NPAD // NS
CH = 80
EDGES_PER_TILE = E // (NC * NS)
N_CHUNK = EDGES_PER_TILE // CH
IB = 25
ZR = 64


def _sc_aggregate(h_user, h_item, e_uci3, e_icu3):
    mesh = plsc.VectorSubcoreMesh(
        core_axis_name="c", subcore_axis_name="s",
        num_cores=NC, num_subcores=NS)

    out_type = (
        jax.ShapeDtypeStruct((NC, NPAD, D), jnp.float32),
        jax.ShapeDtypeStruct((NC, NPAD, LANES), jnp.float32),
        jax.ShapeDtypeStruct((NC, NPAD, D), jnp.float32),
        jax.ShapeDtypeStruct((NC, NPAD, LANES), jnp.float32),
    )

    @functools.partial(
        pl.kernel, mesh=mesh, out_type=out_type,
        compiler_params=pltpu.CompilerParams(use_tc_tiling_on_sc=False),
        scratch_types=[
            pltpu.VMEM_SHARED((NPAD, D), jnp.float32),
            pltpu.VMEM_SHARED((NPAD, LANES), jnp.float32),
            pltpu.VMEM((IB, CH), jnp.int32),
            pltpu.VMEM((IB, CH), jnp.int32),
            pltpu.VMEM((2, CH, D), jnp.float32),
            pltpu.VMEM((CH, LANES), jnp.float32),
            pltpu.VMEM((ZR, D), jnp.float32),
            pltpu.VMEM((ZR, LANES), jnp.float32),
            pltpu.SemaphoreType.DMA,
            pltpu.SemaphoreType.DMA,
            pltpu.SemaphoreType.DMA,
        ],
    )
    def sc_kernel(hu, hi, e_uci, e_icu,
                  agg_item, cnt_item, agg_user, cnt_user,
                  agg_sh, cnt_sh, sidx_v, didx_v, rows_v,
                  ones_v, zagg_v, zcnt_v, gsemA, gsemB, ssem):
        c = lax.axis_index("c")
        s = lax.axis_index("s")
        row0 = s * ROWS_PER_TILE

        zeros16 = jnp.zeros((LANES,), jnp.float32)
        ones16 = jnp.ones((LANES,), jnp.float32)

        def _fill_rows(ref, nrows, val):
            def row_body(r, _):
                def col_body(cc, __):
                    ref[r, pl.ds(cc * LANES, LANES)] = val
                    return __
                return lax.fori_loop(0, ref.shape[1] // LANES, col_body, _)
            lax.fori_loop(0, nrows, row_body, 0)

        _fill_rows(zagg_v, ZR, zeros16)
        _fill_rows(zcnt_v, ZR, zeros16)
        _fill_rows(ones_v, CH, ones16)

        for h, edges, agg_out, cnt_out in (
                (hu, e_uci, agg_item, cnt_item),
                (hi, e_icu, agg_user, cnt_user)):
            zcopies = []
            for j in range(ROWS_PER_TILE // ZR):
                zcopies.append(pltpu.async_copy(
                    zagg_v, agg_sh.at[pl.ds(row0 + j * ZR, ZR)], gsemA))
                zcopies.append(pltpu.async_copy(
                    zcnt_v, cnt_sh.at[pl.ds(row0 + j * ZR, ZR)], gsemA))
            for dsc in zcopies:
                dsc.wait()
            plsc.subcore_barrier()

            r0 = c * (E // NC // CH) + s * N_CHUNK

            def scatter_pair(i, p):
                return ((rows_v.at[p], agg_sh.at[didx_v.at[i]]),
                        (ones_v, cnt_sh.at[didx_v.at[i]]))

            def drain_scatters(i, p):
                (rs, rd), (os_, od) = scatter_pair(i, p)
                pltpu.make_async_copy(rs, rd, ssem).wait()
                pltpu.make_async_copy(os_, od, ssem).wait()

            def issue_scatters(i, p):
                (rs, rd), (os_, od) = scatter_pair(i, p)
                pltpu.async_copy(rs, rd, ssem, add=True)
                pltpu.async_copy(os_, od, ssem, add=True)

            def block_body(blk, carry):
                brow = r0 + blk * IB
                pltpu.sync_copy(edges.at[0, pl.ds(brow, IB)], sidx_v)
                pltpu.sync_copy(edges.at[1, pl.ds(brow, IB)], didx_v)

                pltpu.async_copy(h.at[sidx_v.at[0]], rows_v.at[0], gsemA)

                def pair_body(k, carry2):
                    i0 = 2 * k
                    i1 = 2 * k + 1

                    @pl.when(k >= 1)
                    def _():
                        drain_scatters(i0 - 1, 1)
                    pltpu.async_copy(h.at[sidx_v.at[i1]], rows_v.at[1], gsemB)
                    pltpu.make_async_copy(h.at[sidx_v.at[i0]],
                                          rows_v.at[0], gsemA).wait()
                    issue_scatters(i0, 0)
                    pltpu.make_async_copy(h.at[sidx_v.at[i1]],
                                          rows_v.at[1], gsemB).wait()
                    drain_scatters(i0, 0)
                    pltpu.async_copy(h.at[sidx_v.at[i0 + 2]],
                                     rows_v.at[0], gsemA)
                    issue_scatters(i1, 1)
                    return carry2

                lax.fori_loop(0, IB // 2, pair_body, 0)

                drain_scatters(jnp.int32(IB - 2), 1)
                pltpu.make_async_copy(h.at[sidx_v.at[IB - 1]],
                                      rows_v.at[0], gsemA).wait()
                issue_scatters(jnp.int32(IB - 1), 0)
                drain_scatters(jnp.int32(IB - 1), 0)
                return carry

            lax.fori_loop(0, N_CHUNK // IB, block_body, 0)
            plsc.subcore_barrier()

            pltpu.sync_copy(agg_sh.at[pl.ds(row0, ROWS_PER_TILE)],
                            agg_out.at[c, pl.ds(row0, ROWS_PER_TILE)])
            pltpu.sync_copy(cnt_sh.at[pl.ds(row0, ROWS_PER_TILE)],
                            cnt_out.at[c, pl.ds(row0, ROWS_PER_TILE)])
            plsc.subcore_barrier()

    return sc_kernel(h_user, h_item, e_uci3, e_icu3)


_BLK = 1000
_GRID = N_NODE // _BLK


def _pre_body(xu_ref, xi_ref, wu_ref, wi_ref, hu_ref, hi_ref):
    dn = (((1,), (1,)), ((), ()))
    hu_ref[...] = lax.dot_general(xu_ref[...], wu_ref[...], dn,
                                  preferred_element_type=jnp.float32)
    hi_ref[...] = lax.dot_general(xi_ref[...], wi_ref[...], dn,
                                  preferred_element_type=jnp.float32)


def _tc_pre(x_user, x_item, w_uci, w_icu):
    blk_x = pl.BlockSpec((_BLK, D), lambda i: (i, 0))
    blk_w = pl.BlockSpec((D, D), lambda i: (0, 0))
    return pl.pallas_call(
        _pre_body,
        grid=(_GRID,),
        in_specs=[blk_x, blk_x, blk_w, blk_w],
        out_specs=[blk_x, blk_x],
        out_shape=[jax.ShapeDtypeStruct((N_NODE, D), jnp.float32)] * 2,
    )(x_user, x_item, w_uci, w_icu)


def _post_body(xu_ref, wu_ref, bsu_ref, bnu_ref, pu_ref, cu_ref,
               xi_ref, wi_ref, bsi_ref, bni_ref, pi_ref, ci_ref,
               ou_ref, oi_ref):
    dn = (((1,), (1,)), ((), ()))
    for x_ref, w_ref, bs_ref, bn_ref, p_ref, c_ref, o_ref in (
            (xu_ref, wu_ref, bsu_ref, bnu_ref, pu_ref, cu_ref, ou_ref),
            (xi_ref, wi_ref, bsi_ref, bni_ref, pi_ref, ci_ref, oi_ref)):
        agg = p_ref[0] + p_ref[1]
        cnt = jnp.maximum(c_ref[0, :, :1] + c_ref[1, :, :1], 1.0)
        self_t = lax.dot_general(x_ref[...], w_ref[...], dn,
                                 preferred_element_type=jnp.float32)
        o_ref[...] = jnp.maximum(
            self_t + bs_ref[...] + agg / cnt + bn_ref[...], 0.0)


def _tc_post(x_user, w_self_user, b_self_user, b_neigh_icu, p_user, c_user,
             x_item, w_self_item, b_self_item, b_neigh_uci, p_item, c_item):
    blk_x = pl.BlockSpec((_BLK, D), lambda i: (i, 0))
    blk_w = pl.BlockSpec((D, D), lambda i: (0, 0))
    blk_b = pl.BlockSpec((1, D), lambda i: (0, 0))
    blk_p = pl.BlockSpec((NC, _BLK, D), lambda i: (0, i, 0))
    blk_c = pl.BlockSpec((NC, _BLK, LANES), lambda i: (0, i, 0))
    return pl.pallas_call(
        _post_body,
        grid=(_GRID,),
        in_specs=[blk_x, blk_w, blk_b, blk_b, blk_p, blk_c,
                  blk_x, blk_w, blk_b, blk_b, blk_p, blk_c],
        out_specs=[blk_x, blk_x],
        out_shape=[jax.ShapeDtypeStruct((N_NODE, D), jnp.float32)] * 2,
    )(x_user, w_self_user, b_self_user, b_neigh_icu, p_user, c_user,
      x_item, w_self_item, b_self_item, b_neigh_uci, p_item, c_item)


def kernel(x_user, x_item, edge_index_user_clicks_item,
           edge_index_item_rev_clicks_user,
           W_self_user, b_self_user, W_self_item, b_self_item,
           W_neigh_uci, b_neigh_uci, W_neigh_icu, b_neigh_icu):
    h_user, h_item = _tc_pre(x_user, x_item, W_neigh_uci, W_neigh_icu)

    agg_item, cnt_item, agg_user, cnt_user = _sc_aggregate(
        h_user, h_item,
        edge_index_user_clicks_item.reshape(2, E // CH, CH),
        edge_index_item_rev_clicks_user.reshape(2, E // CH, CH))

    out_user, out_item = _tc_post(
        x_user, W_self_user, b_self_user.reshape(1, D), b_neigh_icu.reshape(1, D),
        agg_user, cnt_user,
        x_item, W_self_item, b_self_item.reshape(1, D), b_neigh_uci.reshape(1, D),
        agg_item, cnt_item)
    return out_user, out_item


# speedup vs baseline: 1.2874x; 1.0186x over previous
import functools

import jax
import jax.numpy as jnp
from jax import lax
from jax.experimental import pallas as pl
from jax.experimental.pallas import tpu as pltpu
from jax.experimental.pallas import tpu_sc as plsc

N_NODE = 10000
E = 320000
D = 128

NC = 2
NS = 16
LANES = 16
NPAD = 10240
ROWS_PER_TILE = NPAD // NS
CH = 80
EDGES_PER_TILE = E // (NC * NS)
N_CHUNK = EDGES_PER_TILE // CH
IB = 25
ZR = 64


def _sc_aggregate(h_user, h_item, e_uci3, e_icu3):
    mesh = plsc.VectorSubcoreMesh(
        core_axis_name="c", subcore_axis_name="s",
        num_cores=NC, num_subcores=NS)

    out_type = (
        jax.ShapeDtypeStruct((NC, NPAD, D), jnp.float32),
        jax.ShapeDtypeStruct((NC, NPAD, LANES), jnp.float32),
        jax.ShapeDtypeStruct((NC, NPAD, D), jnp.float32),
        jax.ShapeDtypeStruct((NC, NPAD, LANES), jnp.float32),
    )

    @functools.partial(
        pl.kernel, mesh=mesh, out_type=out_type,
        compiler_params=pltpu.CompilerParams(use_tc_tiling_on_sc=False),
        scratch_types=[
            pltpu.VMEM_SHARED((NPAD, D), jnp.float32),
            pltpu.VMEM_SHARED((NPAD, LANES), jnp.float32),
            pltpu.VMEM((2, IB, CH), jnp.int32),
            pltpu.VMEM((2, CH, D), jnp.float32),
            pltpu.VMEM((CH, LANES), jnp.float32),
            pltpu.VMEM((ZR, D), jnp.float32),
            pltpu.VMEM((ZR, LANES), jnp.float32),
            pltpu.SemaphoreType.DMA,
            pltpu.SemaphoreType.DMA,
            pltpu.SemaphoreType.DMA,
        ],
    )
    def sc_kernel(hu, hi, e_uci, e_icu,
                  agg_item, cnt_item, agg_user, cnt_user,
                  agg_sh, cnt_sh, eidx_v, rows_v,
                  ones_v, zagg_v, zcnt_v, gsemA, gsemB, ssem):
        c = lax.axis_index("c")
        s = lax.axis_index("s")
        row0 = s * ROWS_PER_TILE

        zeros16 = jnp.zeros((LANES,), jnp.float32)
        ones16 = jnp.ones((LANES,), jnp.float32)

        def _fill_rows(ref, nrows, val):
            def row_body(r, _):
                def col_body(cc, __):
                    ref[r, pl.ds(cc * LANES, LANES)] = val
                    return __
                return lax.fori_loop(0, ref.shape[1] // LANES, col_body, _)
            lax.fori_loop(0, nrows, row_body, 0)

        _fill_rows(zagg_v, ZR, zeros16)
        _fill_rows(zcnt_v, ZR, zeros16)
        _fill_rows(ones_v, CH, ones16)

        for h, edges, agg_out, cnt_out in (
                (hu, e_uci, agg_item, cnt_item),
                (hi, e_icu, agg_user, cnt_user)):
            zcopies = []
            for j in range(ROWS_PER_TILE // ZR):
                zcopies.append(pltpu.async_copy(
                    zagg_v, agg_sh.at[pl.ds(row0 + j * ZR, ZR)], gsemA))
                zcopies.append(pltpu.async_copy(
                    zcnt_v, cnt_sh.at[pl.ds(row0 + j * ZR, ZR)], gsemA))
            for dsc in zcopies:
                dsc.wait()
            plsc.subcore_barrier()

            r0 = c * (E // NC // CH) + s * N_CHUNK

            def scatter_pair(i, p):
                return ((rows_v.at[p], agg_sh.at[eidx_v.at[1, i]]),
                        (ones_v, cnt_sh.at[eidx_v.at[1, i]]))

            def drain_scatters(i, p):
                (rs, rd), (os_, od) = scatter_pair(i, p)
                pltpu.make_async_copy(rs, rd, ssem).wait()
                pltpu.make_async_copy(os_, od, ssem).wait()

            def issue_scatters(i, p):
                (rs, rd), (os_, od) = scatter_pair(i, p)
                pltpu.async_copy(rs, rd, ssem, add=True)
                pltpu.async_copy(os_, od, ssem, add=True)

            def block_body(blk, carry):
                brow = r0 + blk * IB
                pltpu.sync_copy(edges.at[:, pl.ds(brow, IB)], eidx_v)

                pltpu.async_copy(h.at[eidx_v.at[0, 0]], rows_v.at[0], gsemA)

                def pair_body(k, carry2):
                    i0 = 2 * k
                    i1 = 2 * k + 1

                    @pl.when(k >= 1)
                    def _():
                        drain_scatters(i0 - 1, 1)
                    pltpu.async_copy(h.at[eidx_v.at[0, i1]], rows_v.at[1], gsemB)
                    pltpu.make_async_copy(h.at[eidx_v.at[0, i0]],
                                          rows_v.at[0], gsemA).wait()
                    issue_scatters(i0, 0)
                    pltpu.make_async_copy(h.at[eidx_v.at[0, i1]],
                                          rows_v.at[1], gsemB).wait()
                    drain_scatters(i0, 0)
                    pltpu.async_copy(h.at[eidx_v.at[0, i0 + 2]],
                                     rows_v.at[0], gsemA)
                    issue_scatters(i1, 1)
                    return carry2

                lax.fori_loop(0, IB // 2, pair_body, 0)

                drain_scatters(jnp.int32(IB - 2), 1)
                pltpu.make_async_copy(h.at[eidx_v.at[0, IB - 1]],
                                      rows_v.at[0], gsemA).wait()
                issue_scatters(jnp.int32(IB - 1), 0)
                drain_scatters(jnp.int32(IB - 1), 0)
                return carry

            lax.fori_loop(0, N_CHUNK // IB, block_body, 0)
            plsc.subcore_barrier()

            pltpu.sync_copy(agg_sh.at[pl.ds(row0, ROWS_PER_TILE)],
                            agg_out.at[c, pl.ds(row0, ROWS_PER_TILE)])
            pltpu.sync_copy(cnt_sh.at[pl.ds(row0, ROWS_PER_TILE)],
                            cnt_out.at[c, pl.ds(row0, ROWS_PER_TILE)])

    return sc_kernel(h_user, h_item, e_uci3, e_icu3)


_BLK = 1000
_GRID = N_NODE // _BLK


def _pre_body(xu_ref, xi_ref, wu_ref, wi_ref, hu_ref, hi_ref):
    dn = (((1,), (1,)), ((), ()))
    hu_ref[...] = lax.dot_general(xu_ref[...], wu_ref[...], dn,
                                  preferred_element_type=jnp.float32)
    hi_ref[...] = lax.dot_general(xi_ref[...], wi_ref[...], dn,
                                  preferred_element_type=jnp.float32)


def _tc_pre(x_user, x_item, w_uci, w_icu):
    blk_x = pl.BlockSpec((_BLK, D), lambda i: (i, 0))
    blk_w = pl.BlockSpec((D, D), lambda i: (0, 0))
    return pl.pallas_call(
        _pre_body,
        grid=(_GRID,),
        in_specs=[blk_x, blk_x, blk_w, blk_w],
        out_specs=[blk_x, blk_x],
        out_shape=[jax.ShapeDtypeStruct((N_NODE, D), jnp.float32)] * 2,
    )(x_user, x_item, w_uci, w_icu)


def _post_body(xu_ref, wu_ref, bsu_ref, bnu_ref, pu_ref, cu_ref,
               xi_ref, wi_ref, bsi_ref, bni_ref, pi_ref, ci_ref,
               ou_ref, oi_ref):
    dn = (((1,), (1,)), ((), ()))
    for x_ref, w_ref, bs_ref, bn_ref, p_ref, c_ref, o_ref in (
            (xu_ref, wu_ref, bsu_ref, bnu_ref, pu_ref, cu_ref, ou_ref),
            (xi_ref, wi_ref, bsi_ref, bni_ref, pi_ref, ci_ref, oi_ref)):
        agg = p_ref[0] + p_ref[1]
        cnt = jnp.maximum(c_ref[0, :, :1] + c_ref[1, :, :1], 1.0)
        self_t = lax.dot_general(x_ref[...], w_ref[...], dn,
                                 preferred_element_type=jnp.float32)
        o_ref[...] = jnp.maximum(
            self_t + bs_ref[...] + agg / cnt + bn_ref[...], 0.0)


def _tc_post(x_user, w_self_user, b_self_user, b_neigh_icu, p_user, c_user,
             x_item, w_self_item, b_self_item, b_neigh_uci, p_item, c_item):
    blk_x = pl.BlockSpec((_BLK, D), lambda i: (i, 0))
    blk_w = pl.BlockSpec((D, D), lambda i: (0, 0))
    blk_b = pl.BlockSpec((1, D), lambda i: (0, 0))
    blk_p = pl.BlockSpec((NC, _BLK, D), lambda i: (0, i, 0))
    blk_c = pl.BlockSpec((NC, _BLK, LANES), lambda i: (0, i, 0))
    return pl.pallas_call(
        _post_body,
        grid=(_GRID,),
        in_specs=[blk_x, blk_w, blk_b, blk_b, blk_p, blk_c,
                  blk_x, blk_w, blk_b, blk_b, blk_p, blk_c],
        out_specs=[blk_x, blk_x],
        out_shape=[jax.ShapeDtypeStruct((N_NODE, D), jnp.float32)] * 2,
    )(x_user, w_self_user, b_self_user, b_neigh_icu, p_user, c_user,
      x_item, w_self_item, b_self_item, b_neigh_uci, p_item, c_item)


def kernel(x_user, x_item, edge_index_user_clicks_item,
           edge_index_item_rev_clicks_user,
           W_self_user, b_self_user, W_self_item, b_self_item,
           W_neigh_uci, b_neigh_uci, W_neigh_icu, b_neigh_icu):
    h_user, h_item = _tc_pre(x_user, x_item, W_neigh_uci, W_neigh_icu)

    agg_item, cnt_item, agg_user, cnt_user = _sc_aggregate(
        h_user, h_item,
        edge_index_user_clicks_item.reshape(2, E // CH, CH),
        edge_index_item_rev_clicks_user.reshape(2, E // CH, CH))

    out_user, out_item = _tc_post(
        x_user, W_self_user, b_self_user.reshape(1, D), b_neigh_icu.reshape(1, D),
        agg_user, cnt_user,
        x_item, W_self_item, b_self_item.reshape(1, D), b_neigh_uci.reshape(1, D),
        agg_item, cnt_item)
    return out_user, out_item
